# baseline (speedup 1.0000x reference)
import functools

import jax
import jax.numpy as jnp
from jax import lax
from jax.experimental import pallas as pl
from jax.experimental.pallas import tpu as pltpu
from jax.experimental.pallas import tpu_sc as plsc

_G = 2


def _wpe_sc_body(ids_hbm, wts_hbm, out_hbm, ids_v, wts_v, row_v, *,
                 E, M, topk, n_workers):
    wid = lax.axis_index("s") * 2 + lax.axis_index("c")
    pltpu.sync_copy(ids_hbm, ids_v)
    pltpu.sync_copy(wts_hbm, wts_v)
    e_per_w = E // n_workers
    for local in range(e_per_w):
        e = wid * e_per_w + local
        for mc in range(M // 16):
            acc = jnp.zeros((16,), jnp.float32)
            for t in range(topk):
                idv = ids_v[pl.ds(t * M + mc * 16, 16)]
                wtv = wts_v[pl.ds(t * M + mc * 16, 16)]
                acc = acc + jnp.where(idv == e, wtv, 0.0)
            row_v[pl.ds(mc * 16, 16)] = acc
        pltpu.sync_copy(row_v, out_hbm.at[e])


def _combine_weights_sc(topk_ids, topk_weights, E):
    M, topk = topk_ids.shape
    n_workers = 32
    mesh = plsc.VectorSubcoreMesh(core_axis_name="c", subcore_axis_name="s")
    fn = functools.partial(
        pl.kernel,
        functools.partial(_wpe_sc_body, E=E, M=M, topk=topk,
                          n_workers=n_workers),
        out_type=jax.ShapeDtypeStruct((E, M), jnp.float32),
        mesh=mesh,
        scratch_types=[
            pltpu.VMEM((M * topk,), jnp.int32),
            pltpu.VMEM((M * topk,), jnp.float32),
            pltpu.VMEM((M,), jnp.float32),
        ],
    )()
    return fn(topk_ids.T.reshape(-1), topk_weights.T.reshape(-1))


def _moe_step(wpe_ref, x_ref, w1g_ref, w1u_ref, w2_ref, out_ref, *, N, G):
    i = pl.program_id(0)
    x = x_ref[...]
    M = x.shape[0]
    ident = (lax.broadcasted_iota(jnp.int32, (M, M), 0)
             == lax.broadcasted_iota(jnp.int32, (M, M), 1)).astype(jnp.float32)
    contrib = None
    for g in range(G):
        gate = jax.lax.dot_general(
            x, w1g_ref[g], (((1,), (1,)), ((), ())),
            preferred_element_type=jnp.float32,
        )
        up = jax.lax.dot_general(
            x, w1u_ref[g], (((1,), (1,)), ((), ())),
            preferred_element_type=jnp.float32,
        )
        act = gate * jax.lax.logistic(gate) * up
        row = wpe_ref[0, g, :].reshape(1, M)
        col = jax.lax.dot_general(
            ident, row, (((1,), (1,)), ((), ())),
            preferred_element_type=jnp.float32,
        )
        act = act * col
        c = jax.lax.dot_general(
            act, w2_ref[g], (((1,), (1,)), ((), ())),
            preferred_element_type=jnp.float32,
        )
        contrib = c if contrib is None else contrib + c

    @pl.when(i == 0)
    def _init():
        out_ref[...] = contrib

    @pl.when(i != 0)
    def _acc():
        out_ref[...] += contrib


def kernel(hidden_states, w1, w2, topk_weights, topk_ids):
    M, K = hidden_states.shape
    E, twoN, _ = w1.shape
    N = twoN // 2
    G = _G
    grid = (E // G,)
    wpe = _combine_weights_sc(topk_ids, topk_weights, E)
    wpe3 = wpe.reshape(E // G, G, M)
    out = pl.pallas_call(
        functools.partial(_moe_step, N=N, G=G),
        grid=grid,
        in_specs=[
            pl.BlockSpec((1, G, M), lambda i: (i, 0, 0)),
            pl.BlockSpec((M, K), lambda i: (0, 0)),
            pl.BlockSpec((G, N, K), lambda i: (i, 0, 0)),
            pl.BlockSpec((G, N, K), lambda i: (i, 1, 0)),
            pl.BlockSpec((G, K, N), lambda i: (i, 0, 0)),
        ],
        out_specs=pl.BlockSpec((M, K), lambda i: (0, 0)),
        out_shape=jax.ShapeDtypeStruct((M, K), hidden_states.dtype),
    )(wpe3, hidden_states, w1, w1, w2)
    return out


# speedup vs baseline: 1.1518x; 1.1518x over previous
import functools

import jax
import jax.numpy as jnp
from jax.experimental import pallas as pl

_G = 2


def _moe_step(ids_ref, wts_ref, x_ref, w1gl_ref, w1gh_ref, w1ul_ref,
              w1uh_ref, w2l_ref, w2h_ref, out_ref, *, N, G):
    i = pl.program_id(0)
    x = x_ref[...]
    ids = ids_ref[...]
    wts = wts_ref[...]
    contrib = None
    for g in range(G):
        e = i * G + g
        wpe = jnp.sum(jnp.where(ids == e, wts, 0.0), axis=1)
        c = None
        for w1g_ref, w1u_ref, w2_ref in (
            (w1gl_ref, w1ul_ref, w2l_ref),
            (w1gh_ref, w1uh_ref, w2h_ref),
        ):
            gate = jax.lax.dot_general(
                x, w1g_ref[g], (((1,), (1,)), ((), ())),
                preferred_element_type=jnp.float32,
            )
            up = jax.lax.dot_general(
                x, w1u_ref[g], (((1,), (1,)), ((), ())),
                preferred_element_type=jnp.float32,
            )
            act = gate * jax.lax.logistic(gate) * up
            act = act * wpe[:, None]
            part = jax.lax.dot_general(
                act, w2_ref[g], (((1,), (1,)), ((), ())),
                preferred_element_type=jnp.float32,
            )
            c = part if c is None else c + part
        contrib = c if contrib is None else contrib + c

    @pl.when(i == 0)
    def _init():
        out_ref[...] = contrib

    @pl.when(i != 0)
    def _acc():
        out_ref[...] += contrib


def kernel(hidden_states, w1, w2, topk_weights, topk_ids):
    M, K = hidden_states.shape
    E, twoN, _ = w1.shape
    N = twoN // 2
    H = N // 2
    G = _G
    grid = (E // G,)
    out = pl.pallas_call(
        functools.partial(_moe_step, N=N, G=G),
        grid=grid,
        in_specs=[
            pl.BlockSpec(topk_ids.shape, lambda i: (0, 0)),
            pl.BlockSpec(topk_weights.shape, lambda i: (0, 0)),
            pl.BlockSpec((M, K), lambda i: (0, 0)),
            pl.BlockSpec((G, H, K), lambda i: (i, 0, 0)),
            pl.BlockSpec((G, H, K), lambda i: (i, 1, 0)),
            pl.BlockSpec((G, H, K), lambda i: (i, 2, 0)),
            pl.BlockSpec((G, H, K), lambda i: (i, 3, 0)),
            pl.BlockSpec((G, K, H), lambda i: (i, 0, 0)),
            pl.BlockSpec((G, K, H), lambda i: (i, 0, 1)),
        ],
        out_specs=pl.BlockSpec((M, K), lambda i: (0, 0)),
        out_shape=jax.ShapeDtypeStruct((M, K), hidden_states.dtype),
    )(topk_ids, topk_weights, hidden_states, w1, w1, w1, w1, w2, w2)
    return out


# speedup vs baseline: 1.1630x; 1.0098x over previous
import functools

import jax
import jax.numpy as jnp
from jax.experimental import pallas as pl

_G = 2
_S = 4


def _moe_step(ids_ref, wts_ref, x_ref, *refs, N, G, S):
    w1g_refs = refs[:S]
    w1u_refs = refs[S:2 * S]
    w2_refs = refs[2 * S:2 * S + 2]
    out_ref = refs[-1]
    i = pl.program_id(0)
    x = x_ref[...]
    ids = ids_ref[...]
    wts = wts_ref[...]
    Hs = N // S
    contrib = None
    for g in range(G):
        e = i * G + g
        wpe = jnp.sum(jnp.where(ids == e, wts, 0.0), axis=1)
        acts = []
        for s in range(S):
            gate = jax.lax.dot_general(
                x, w1g_refs[s][g], (((1,), (1,)), ((), ())),
                preferred_element_type=jnp.float32,
            )
            up = jax.lax.dot_general(
                x, w1u_refs[s][g], (((1,), (1,)), ((), ())),
                preferred_element_type=jnp.float32,
            )
            act = gate * jax.lax.logistic(gate) * up
            acts.append(act * wpe[:, None])
        c = None
        for h in range(2):
            act_h = jnp.concatenate(acts[h * (S // 2):(h + 1) * (S // 2)],
                                    axis=1)
            part = jax.lax.dot_general(
                act_h, w2_refs[h][g], (((1,), (1,)), ((), ())),
                preferred_element_type=jnp.float32,
            )
            c = part if c is None else c + part
        contrib = c if contrib is None else contrib + c

    @pl.when(i == 0)
    def _init():
        out_ref[...] = contrib

    @pl.when(i != 0)
    def _acc():
        out_ref[...] += contrib


def kernel(hidden_states, w1, w2, topk_weights, topk_ids):
    M, K = hidden_states.shape
    E, twoN, _ = w1.shape
    N = twoN // 2
    G = _G
    S = _S
    Hs = N // S
    grid = (E // G,)
    w1_specs = []
    for half in range(2):
        for s in range(S):
            blk = half * S + s
            w1_specs.append(
                pl.BlockSpec((G, Hs, K), lambda i, blk=blk: (i, blk, 0)))
    w2_specs = [
        pl.BlockSpec((G, K, N // 2), lambda i: (i, 0, 0)),
        pl.BlockSpec((G, K, N // 2), lambda i: (i, 0, 1)),
    ]
    out = pl.pallas_call(
        functools.partial(_moe_step, N=N, G=G, S=S),
        grid=grid,
        in_specs=[
            pl.BlockSpec(topk_ids.shape, lambda i: (0, 0)),
            pl.BlockSpec(topk_weights.shape, lambda i: (0, 0)),
            pl.BlockSpec((M, K), lambda i: (0, 0)),
            *w1_specs,
            *w2_specs,
        ],
        out_specs=pl.BlockSpec((M, K), lambda i: (0, 0)),
        out_shape=jax.ShapeDtypeStruct((M, K), hidden_states.dtype),
    )(topk_ids, topk_weights, hidden_states,
      *([w1] * (2 * S)), *([w2] * 2))
    return out
